# baseline (speedup 1.0000x reference)
import functools

import jax
import jax.numpy as jnp
from jax import lax
from jax.experimental import pallas as pl
from jax.experimental.pallas import tpu as pltpu


def _router_kernel(T, NB, x_ref, gw_ref, gb_ref,
                   rw_ref, be_ref, bs_ref, bv_ref):
    S, Hd = x_ref.shape
    E = gw_ref.shape[1]
    f32 = jnp.float32
    bf16 = jnp.bfloat16

    logits = lax.dot(x_ref[...], gw_ref[...].astype(bf16),
                     preferred_element_type=f32) + gb_ref[...]

    m1 = jnp.max(logits, axis=1, keepdims=True)
    ex = jnp.exp(logits - m1)
    z = jnp.sum(ex, axis=1, keepdims=True)
    p = ex / z
    ioe = lax.broadcasted_iota(jnp.int32, (S, E), 1)
    pm1 = jnp.max(p, axis=1, keepdims=True)
    i1 = jnp.min(jnp.where(p == pm1, ioe, E), axis=1, keepdims=True)
    p_m = jnp.where(ioe == i1, -1.0, p)
    pm2 = jnp.max(p_m, axis=1, keepdims=True)
    i2 = jnp.min(jnp.where(p_m == pm2, ioe, E), axis=1, keepdims=True)
    denom = pm1 + pm2 + 1e-6
    sel1 = ioe == i1
    sel2 = ioe == i2
    wc = jnp.where(sel1, pm1 / denom, 0.0) + jnp.where(sel2, pm2 / denom, 0.0)
    m = jnp.logical_or(sel1, sel2).astype(f32)

    rio = lax.broadcasted_iota(jnp.int32, (S, S), 0)
    cio = lax.broadcasted_iota(jnp.int32, (S, S), 1)
    ltri = (cio < rio).astype(bf16)
    rank = lax.dot(ltri, m.astype(bf16), preferred_element_type=f32)
    rw_ref[...] = jnp.concatenate(
        [jnp.where(m > 0.5, rank, -1.0), wc], axis=1)

    ones_col = jnp.ones((S, 1), bf16)
    counts_t = lax.dot_general(m.astype(bf16), ones_col,
                               (((0,), (0,)), ((), ())),
                               preferred_element_type=f32)
    nb_t = jnp.floor((counts_t + (T - 1)) / T)
    eio_r = lax.broadcasted_iota(jnp.int32, (E, E), 0)
    eio_c = lax.broadcasted_iota(jnp.int32, (E, E), 1)
    l8 = (eio_r > eio_c).astype(bf16)
    pad_t = lax.dot(l8, nb_t.astype(bf16), preferred_element_type=f32)
    total = jnp.sum(nb_t, axis=0, keepdims=True)

    bio = lax.broadcasted_iota(jnp.int32, (E, NB), 1).astype(f32)
    bsafe = jnp.minimum(bio, total - 1.0)
    cmp = (pad_t <= bsafe).astype(f32)
    be_row = jnp.sum(cmp, axis=0, keepdims=True) - 1.0
    pad_sel = jnp.max(jnp.where(cmp > 0.5, jnp.broadcast_to(pad_t, (E, NB)), 0.0),
                      axis=0, keepdims=True)
    bio1 = lax.broadcasted_iota(jnp.int32, (1, NB), 1).astype(f32)
    bsafe1 = jnp.minimum(bio1, total - 1.0)
    bs_row = (bsafe1 - pad_sel) * T
    be_ref[...] = be_row.astype(jnp.int32)
    bs_ref[...] = bs_row.astype(jnp.int32)
    bv_ref[...] = (bio1 < total).astype(jnp.int32)


def _ffn_kernel(T, KI, be_ref, bs_ref, bv_ref,
                x_ref, rw_ref, w1_ref, b1_ref, w2_ref, b2_ref,
                out_ref, pt_ref, xs_ref, ya_ref):
    S, Hd = x_ref.shape
    E = rw_ref.shape[1] // 2
    f32 = jnp.float32
    bf16 = jnp.bfloat16
    b = pl.program_id(0)
    ki = pl.program_id(1)

    @pl.when(jnp.logical_and(b == 0, ki == 0))
    def _init():
        out_ref[...] = jnp.zeros_like(out_ref)

    valid = bv_ref[b] == 1
    e = be_ref[b]

    @pl.when(valid)
    def _body():
        @pl.when(ki == 0)
        def _gather():
            ioe = lax.broadcasted_iota(jnp.int32, (S, E), 1)
            sel = ioe == e
            r = jnp.sum(jnp.where(sel, rw_ref[:, :E], 0.0), axis=1, keepdims=True)
            tio = lax.broadcasted_iota(jnp.int32, (S, T), 1).astype(f32)
            startf = bs_ref[b].astype(f32)
            pt_ref[...] = (r == tio + startf).astype(bf16)
            xs = lax.dot_general(pt_ref[...], x_ref[...],
                                 (((0,), (0,)), ((), ())),
                                 preferred_element_type=f32)
            xs_ref[...] = xs.astype(bf16)

        h = lax.dot(xs_ref[...], w1_ref[0].astype(bf16),
                    preferred_element_type=f32) + b1_ref[0]
        h = jax.nn.gelu(h, approximate=True)
        part = lax.dot(h.astype(bf16), w2_ref[0].astype(bf16),
                       preferred_element_type=f32)

        @pl.when(ki == 0)
        def _acc0():
            ya_ref[...] = part

        @pl.when(ki != 0)
        def _accn():
            ya_ref[...] += part

        @pl.when(ki == KI - 1)
        def _combine():
            ioe = lax.broadcasted_iota(jnp.int32, (S, E), 1)
            wv = jnp.sum(jnp.where(ioe == e, rw_ref[:, E:], 0.0),
                         axis=1, keepdims=True)
            wsel = lax.dot_general(pt_ref[...], wv.astype(bf16),
                                   (((0,), (0,)), ((), ())),
                                   preferred_element_type=f32)
            y = ((ya_ref[...] + b2_ref[0]) * wsel).astype(bf16)
            out_ref[...] += lax.dot(pt_ref[...], y,
                                    preferred_element_type=f32)


def kernel(x, gate_w, gate_b, w1, b1, w2, b2):
    Bx, Sx, Hd = x.shape
    E = gate_w.shape[1]
    I = w1.shape[2]
    S = Bx * Sx
    T = 576
    NB = -((-2 * S) // T) + (E - 1)
    TI = 1024
    KI = I // TI

    flat = x.reshape(S, Hd).astype(jnp.bfloat16)
    gb2 = gate_b.reshape(1, E)

    rw, be, bs, bv = pl.pallas_call(
        functools.partial(_router_kernel, T, NB),
        out_shape=[
            jax.ShapeDtypeStruct((S, 2 * E), jnp.float32),
            jax.ShapeDtypeStruct((1, NB), jnp.int32),
            jax.ShapeDtypeStruct((1, NB), jnp.int32),
            jax.ShapeDtypeStruct((1, NB), jnp.int32),
        ],
    )(flat, gate_w, gb2)

    be = be.reshape(NB)
    bs = bs.reshape(NB)
    bv = bv.reshape(NB)

    def _clamped_ki(b_i, ki_i, bv_s):
        return jnp.where(bv_s[b_i] == 1, ki_i, KI - 1)

    def w1_map(b_i, ki_i, be_s, bs_s, bv_s):
        return (be_s[b_i], 0, _clamped_ki(b_i, ki_i, bv_s))

    def b1_map(b_i, ki_i, be_s, bs_s, bv_s):
        return (be_s[b_i] * KI + _clamped_ki(b_i, ki_i, bv_s), 0, 0)

    def w2_map(b_i, ki_i, be_s, bs_s, bv_s):
        return (be_s[b_i], _clamped_ki(b_i, ki_i, bv_s), 0)

    def b2_map(b_i, ki_i, be_s, bs_s, bv_s):
        return (be_s[b_i], 0, 0)

    grid_spec = pltpu.PrefetchScalarGridSpec(
        num_scalar_prefetch=3,
        grid=(NB, KI),
        in_specs=[
            pl.BlockSpec((S, Hd), lambda b_i, ki_i, *_: (0, 0)),
            pl.BlockSpec((S, 2 * E), lambda b_i, ki_i, *_: (0, 0)),
            pl.BlockSpec((1, Hd, TI), w1_map),
            pl.BlockSpec((1, 1, TI), b1_map),
            pl.BlockSpec((1, TI, Hd), w2_map),
            pl.BlockSpec((1, 1, Hd), b2_map),
        ],
        out_specs=pl.BlockSpec((S, Hd), lambda b_i, ki_i, *_: (0, 0)),
        scratch_shapes=[
            pltpu.VMEM((S, T), jnp.bfloat16),
            pltpu.VMEM((T, Hd), jnp.bfloat16),
            pltpu.VMEM((T, Hd), jnp.float32),
        ],
    )

    out = pl.pallas_call(
        functools.partial(_ffn_kernel, T, KI),
        grid_spec=grid_spec,
        out_shape=jax.ShapeDtypeStruct((S, Hd), jnp.float32),
    )(be, bs, bv, flat, rw,
      w1, b1.reshape(E * KI, 1, TI), w2, b2.reshape(E, 1, Hd))

    return out.reshape(Bx, Sx, Hd)


# speedup vs baseline: 1.1025x; 1.1025x over previous
import functools

import jax
import jax.numpy as jnp
from jax import lax
from jax.experimental import pallas as pl
from jax.experimental.pallas import tpu as pltpu


def _router_kernel(T, NB, x_ref, gw_ref, gb_ref,
                   rank_ref, wc_ref, be_ref, bs_ref, bv_ref):
    S, Hd = x_ref.shape
    E = gw_ref.shape[1]
    f32 = jnp.float32
    bf16 = jnp.bfloat16

    logits = lax.dot(x_ref[...], gw_ref[...].astype(bf16),
                     preferred_element_type=f32) + gb_ref[...]

    m1 = jnp.max(logits, axis=1, keepdims=True)
    ex = jnp.exp(logits - m1)
    z = jnp.sum(ex, axis=1, keepdims=True)
    p = ex / z
    ioe = lax.broadcasted_iota(jnp.int32, (S, E), 1)
    pm1 = jnp.max(p, axis=1, keepdims=True)
    i1 = jnp.min(jnp.where(p == pm1, ioe, E), axis=1, keepdims=True)
    p_m = jnp.where(ioe == i1, -1.0, p)
    pm2 = jnp.max(p_m, axis=1, keepdims=True)
    i2 = jnp.min(jnp.where(p_m == pm2, ioe, E), axis=1, keepdims=True)
    denom = pm1 + pm2 + 1e-6
    sel1 = ioe == i1
    sel2 = ioe == i2
    wc = jnp.where(sel1, pm1 / denom, 0.0) + jnp.where(sel2, pm2 / denom, 0.0)
    m = jnp.logical_or(sel1, sel2).astype(f32)

    rio = lax.broadcasted_iota(jnp.int32, (S, S), 0)
    cio = lax.broadcasted_iota(jnp.int32, (S, S), 1)
    ltri = (cio < rio).astype(bf16)
    rank = lax.dot(ltri, m.astype(bf16), preferred_element_type=f32)
    rank_ref[...] = jnp.where(m > 0.5, rank, -1.0)
    wc_ref[...] = wc

    ones_col = jnp.ones((S, 1), bf16)
    counts_t = lax.dot_general(m.astype(bf16), ones_col,
                               (((0,), (0,)), ((), ())),
                               preferred_element_type=f32)
    nb_t = jnp.floor((counts_t + (T - 1)) / T)
    eio_r = lax.broadcasted_iota(jnp.int32, (E, E), 0)
    eio_c = lax.broadcasted_iota(jnp.int32, (E, E), 1)
    l8 = (eio_r > eio_c).astype(bf16)
    pad_t = lax.dot(l8, nb_t.astype(bf16), preferred_element_type=f32)
    total = jnp.sum(nb_t, axis=0, keepdims=True)

    bio = lax.broadcasted_iota(jnp.int32, (E, NB), 1).astype(f32)
    bsafe = jnp.minimum(bio, total - 1.0)
    cmp = (pad_t <= bsafe).astype(f32)
    be_row = jnp.sum(cmp, axis=0, keepdims=True) - 1.0
    pad_sel = jnp.max(jnp.where(cmp > 0.5, jnp.broadcast_to(pad_t, (E, NB)), 0.0),
                      axis=0, keepdims=True)
    bio1 = lax.broadcasted_iota(jnp.int32, (1, NB), 1).astype(f32)
    bsafe1 = jnp.minimum(bio1, total - 1.0)
    bs_row = (bsafe1 - pad_sel) * T
    be_ref[...] = be_row.astype(jnp.int32)
    bs_ref[...] = bs_row.astype(jnp.int32)
    bv_ref[...] = (bio1 < total).astype(jnp.int32)


def _ffn_kernel(T, KI, be_ref, bs_ref, bv_ref,
                x_ref, rank_ref, wc_ref, w1_ref, b1_ref, w2_ref, b2_ref,
                out_ref, pt_ref, xs_ref, ya_ref):
    S, Hd = x_ref.shape
    E = rank_ref.shape[1]
    f32 = jnp.float32
    bf16 = jnp.bfloat16
    b = pl.program_id(0)
    ki = pl.program_id(1)

    @pl.when(jnp.logical_and(b == 0, ki == 0))
    def _init():
        out_ref[...] = jnp.zeros_like(out_ref)

    valid = bv_ref[b] == 1
    e = be_ref[b]

    @pl.when(valid)
    def _body():
        @pl.when(ki == 0)
        def _gather():
            ioe = lax.broadcasted_iota(jnp.int32, (S, E), 1)
            sel = ioe == e
            r = jnp.sum(jnp.where(sel, rank_ref[...], 0.0), axis=1, keepdims=True)
            tio = lax.broadcasted_iota(jnp.int32, (S, T), 1).astype(f32)
            startf = bs_ref[b].astype(f32)
            pt_ref[...] = (r == tio + startf).astype(bf16)
            xs = lax.dot_general(pt_ref[...], x_ref[...],
                                 (((0,), (0,)), ((), ())),
                                 preferred_element_type=f32)
            xs_ref[...] = xs.astype(bf16)

        h = lax.dot(xs_ref[...], w1_ref[0].astype(bf16),
                    preferred_element_type=f32) + b1_ref[0]
        h = jax.nn.gelu(h, approximate=True)
        part = lax.dot(h.astype(bf16), w2_ref[0].astype(bf16),
                       preferred_element_type=f32)

        @pl.when(ki == 0)
        def _acc0():
            ya_ref[...] = part

        @pl.when(ki != 0)
        def _accn():
            ya_ref[...] += part

        @pl.when(ki == KI - 1)
        def _combine():
            y = (ya_ref[...] + b2_ref[0]).astype(bf16)
            res = lax.dot(pt_ref[...], y, preferred_element_type=f32)
            ioe = lax.broadcasted_iota(jnp.int32, (S, E), 1)
            wv = jnp.sum(jnp.where(ioe == e, wc_ref[...], 0.0),
                         axis=1, keepdims=True)
            out_ref[...] += wv * res


def kernel(x, gate_w, gate_b, w1, b1, w2, b2):
    Bx, Sx, Hd = x.shape
    E = gate_w.shape[1]
    I = w1.shape[2]
    S = Bx * Sx
    T = 576
    NB = -((-2 * S) // T) + (E - 1)
    TI = 1024
    KI = I // TI

    flat = x.reshape(S, Hd).astype(jnp.bfloat16)
    gb2 = gate_b.reshape(1, E)

    rank, wc, be, bs, bv = pl.pallas_call(
        functools.partial(_router_kernel, T, NB),
        out_shape=[
            jax.ShapeDtypeStruct((S, E), jnp.float32),
            jax.ShapeDtypeStruct((S, E), jnp.float32),
            jax.ShapeDtypeStruct((1, NB), jnp.int32),
            jax.ShapeDtypeStruct((1, NB), jnp.int32),
            jax.ShapeDtypeStruct((1, NB), jnp.int32),
        ],
    )(flat, gate_w, gb2)

    be = be.reshape(NB)
    bs = bs.reshape(NB)
    bv = bv.reshape(NB)

    def _clamped_ki(b_i, ki_i, bv_s):
        return jnp.where(bv_s[b_i] == 1, ki_i, KI - 1)

    def w1_map(b_i, ki_i, be_s, bs_s, bv_s):
        return (be_s[b_i], 0, _clamped_ki(b_i, ki_i, bv_s))

    def b1_map(b_i, ki_i, be_s, bs_s, bv_s):
        return (be_s[b_i] * KI + _clamped_ki(b_i, ki_i, bv_s), 0, 0)

    def w2_map(b_i, ki_i, be_s, bs_s, bv_s):
        return (be_s[b_i], _clamped_ki(b_i, ki_i, bv_s), 0)

    def b2_map(b_i, ki_i, be_s, bs_s, bv_s):
        return (be_s[b_i], 0, 0)

    grid_spec = pltpu.PrefetchScalarGridSpec(
        num_scalar_prefetch=3,
        grid=(NB, KI),
        in_specs=[
            pl.BlockSpec((S, Hd), lambda b_i, ki_i, *_: (0, 0)),
            pl.BlockSpec((S, E), lambda b_i, ki_i, *_: (0, 0)),
            pl.BlockSpec((S, E), lambda b_i, ki_i, *_: (0, 0)),
            pl.BlockSpec((1, Hd, TI), w1_map),
            pl.BlockSpec((1, 1, TI), b1_map),
            pl.BlockSpec((1, TI, Hd), w2_map),
            pl.BlockSpec((1, 1, Hd), b2_map),
        ],
        out_specs=pl.BlockSpec((S, Hd), lambda b_i, ki_i, *_: (0, 0)),
        scratch_shapes=[
            pltpu.VMEM((S, T), jnp.bfloat16),
            pltpu.VMEM((T, Hd), jnp.bfloat16),
            pltpu.VMEM((T, Hd), jnp.float32),
        ],
    )

    out = pl.pallas_call(
        functools.partial(_ffn_kernel, T, KI),
        grid_spec=grid_spec,
        out_shape=jax.ShapeDtypeStruct((S, Hd), jnp.float32),
    )(be, bs, bv, flat, rank, wc,
      w1, b1.reshape(E * KI, 1, TI), w2, b2.reshape(E, 1, Hd))

    return out.reshape(Bx, Sx, Hd)
